# baseline (speedup 1.0000x reference)
import numpy as np

import jax
import jax.numpy as jnp
from jax.experimental import pallas as pl
from jax.experimental.pallas import tpu as pltpu
from jax.sharding import Mesh, PartitionSpec as P

HIDDEN = 128
BLOCK_ROWS = 256


def _broadcast_body(table_ref, out_ref, scratch, sem):
    scratch[...] = jnp.broadcast_to(table_ref[...], scratch.shape)
    batch = out_ref.shape[0]
    copies = []
    for t in range(batch // BLOCK_ROWS):
        c = pltpu.make_async_copy(
            scratch, out_ref.at[pl.ds(t * BLOCK_ROWS, BLOCK_ROWS)], sem
        )
        c.start()
        copies.append(c)
    for c in copies:
        c.wait()


def _broadcast_rows(table, rows):
    return pl.pallas_call(
        _broadcast_body,
        in_specs=[pl.BlockSpec(memory_space=pltpu.VMEM)],
        out_specs=pl.BlockSpec(memory_space=pl.ANY),
        out_shape=jax.ShapeDtypeStruct((rows, HIDDEN), jnp.float32),
        scratch_shapes=[
            pltpu.VMEM((BLOCK_ROWS, HIDDEN), jnp.float32),
            pltpu.SemaphoreType.DMA,
        ],
    )(table)


def kernel(x, table):
    batch = x.shape[0]
    table = table.astype(jnp.float32)
    devs = jax.devices()
    ndev = len(devs)
    if ndev > 1 and batch % (ndev * BLOCK_ROWS) == 0:
        shard = jax.shard_map(
            lambda t: _broadcast_rows(t, batch // ndev),
            mesh=Mesh(np.array(devs), ("b",)),
            in_specs=P(None, None),
            out_specs=P("b", None),
            check_vma=False,
        )
        return shard(table)
    return _broadcast_rows(table, batch)


# speedup vs baseline: 34.0412x; 34.0412x over previous
import jax
import jax.numpy as jnp
from jax.experimental import pallas as pl
from jax.experimental.pallas import tpu as pltpu

HIDDEN = 128
BLOCK_ROWS = 128


def _broadcast_body(table_ref, out_ref, scratch, sem):
    scratch[...] = jnp.broadcast_to(table_ref[...], scratch.shape)
    batch = out_ref.shape[0]
    copies = []
    for t in range(batch // BLOCK_ROWS):
        c = pltpu.make_async_copy(
            scratch, out_ref.at[pl.ds(t * BLOCK_ROWS, BLOCK_ROWS)], sem
        )
        c.start()
        copies.append(c)
    for c in copies:
        c.wait()


def kernel(x, table):
    batch = x.shape[0]
    return pl.pallas_call(
        _broadcast_body,
        in_specs=[pl.BlockSpec(memory_space=pltpu.VMEM)],
        out_specs=pl.BlockSpec(memory_space=pl.ANY),
        out_shape=jax.ShapeDtypeStruct((batch, HIDDEN), jnp.float32),
        scratch_shapes=[
            pltpu.VMEM((BLOCK_ROWS, HIDDEN), jnp.float32),
            pltpu.SemaphoreType.DMA,
        ],
    )(table.astype(jnp.float32))


# speedup vs baseline: 34.1779x; 1.0040x over previous
import jax
import jax.numpy as jnp
from jax.experimental import pallas as pl
from jax.experimental.pallas import tpu as pltpu

HIDDEN = 128
BLOCK_ROWS = 256


def _broadcast_body(table_ref, out_ref, scratch, sem):
    scratch[...] = jnp.broadcast_to(table_ref[...], scratch.shape)
    batch = out_ref.shape[0]
    copies = []
    for t in range(batch // BLOCK_ROWS):
        c = pltpu.make_async_copy(
            scratch, out_ref.at[pl.ds(t * BLOCK_ROWS, BLOCK_ROWS)], sem
        )
        c.start()
        copies.append(c)
    for c in copies:
        c.wait()


def kernel(x, table):
    batch = x.shape[0]
    return pl.pallas_call(
        _broadcast_body,
        in_specs=[pl.BlockSpec(memory_space=pltpu.VMEM)],
        out_specs=pl.BlockSpec(memory_space=pl.ANY),
        out_shape=jax.ShapeDtypeStruct((batch, HIDDEN), jnp.float32),
        scratch_shapes=[
            pltpu.VMEM((BLOCK_ROWS, HIDDEN), jnp.float32),
            pltpu.SemaphoreType.DMA,
        ],
    )(table.astype(jnp.float32))
